# Initial kernel scaffold; baseline (speedup 1.0000x reference)
import jax
import jax.numpy as jnp
from jax.experimental import pallas as pl


def kernel(embedding, hidden_states, embedding_bias, temperatures, top_ps, top_ks):
    raise NotImplementedError("write your pallas kernel here")


# speedup vs baseline: 14.2826x; 14.2826x over previous
import math

import jax
import jax.numpy as jnp
from jax.experimental import pallas as pl
from jax.experimental.pallas import tpu as pltpu

_V = 100000
_D = 1024
_BLK = 2048
_NBLK = (_V + _BLK - 1) // _BLK
_VPAD = _NBLK * _BLK
_EPS = 1e-05
_NEG = -1e30
_LOG_TINY = math.log(1e-30)
_T_BISECT = 42


def _body(h_ref, bias_ref, t_ref, tp_ref, tk_ref, g_ref, emb_ref,
          ids_ref, lp_ref, l_scr, e_scr):
    i = pl.program_id(0)
    h = h_ref[...]
    emb = emb_ref[...]
    lg = jax.lax.dot_general(h, emb, (((1,), (1,)), ((), ())),
                             preferred_element_type=jnp.float32)
    t = t_ref[...]
    t = jnp.where(t < _EPS, 1.0, t)
    lg = (lg + bias_ref[...]) / t
    col = i * _BLK + jax.lax.broadcasted_iota(jnp.int32, lg.shape, 1)
    lg = jnp.where(col < _V, lg, _NEG)
    l_scr[:, pl.ds(i * _BLK, _BLK)] = lg

    @pl.when(i == _NBLK - 1)
    def _finalize():
        l = l_scr[...]
        m = jnp.max(l, axis=1, keepdims=True)
        e = jnp.exp(l - m)
        e_scr[...] = e
        s = jnp.sum(e, axis=1, keepdims=True)
        colf = jax.lax.broadcasted_iota(jnp.int32, l.shape, 1)
        vmin = jnp.min(jnp.where(colf < _V, l, jnp.inf), axis=1, keepdims=True)
        lo0 = vmin - 1.0

        tkf = tk_ref[...].astype(jnp.float32)
        tps_s = tp_ref[...] * s

        def bis(_, c):
            lo_k, hi_k, lo_p, hi_p = c
            mid_k = 0.5 * (lo_k + hi_k)
            mid_p = 0.5 * (lo_p + hi_p)
            ll = l_scr[...]
            ee = e_scr[...]
            cnt = jnp.sum(jnp.where(ll > mid_k, 1.0, 0.0), axis=1,
                          keepdims=True)
            mass = jnp.sum(jnp.where(ll > mid_p, ee, 0.0), axis=1,
                           keepdims=True)
            ok_k = cnt < tkf
            ok_p = mass <= tps_s
            return (jnp.where(ok_k, lo_k, mid_k), jnp.where(ok_k, mid_k, hi_k),
                    jnp.where(ok_p, lo_p, mid_p), jnp.where(ok_p, mid_p, hi_p))

        _, hi_k, _, hi_p = jax.lax.fori_loop(
            0, _T_BISECT, bis, (lo0, m, lo0, m))
        tau = jnp.maximum(hi_k, hi_p)

        kept = l >= tau
        g = g_ref[...]
        z = jnp.where(kept, jnp.log(e / s + 1e-30), _LOG_TINY) + g
        zmax = jnp.max(z, axis=1, keepdims=True)
        idx = jnp.min(jnp.where(z == zmax, colf, _VPAD), axis=1, keepdims=True)
        lsel = jnp.max(jnp.where(colf == idx, l, _NEG), axis=1, keepdims=True)
        ids_ref[...] = idx
        lp_ref[...] = lsel - m - jnp.log(s)


def kernel(embedding, hidden_states, embedding_bias, temperatures, top_ps,
           top_ks):
    if hidden_states.ndim == 1:
        hidden_states = hidden_states.reshape(1, -1)
    b = hidden_states.shape[0]
    g = jax.random.gumbel(jax.random.key(1234), (b, _V), jnp.float32)
    g = jnp.pad(g, ((0, 0), (0, _VPAD - _V)))
    bias2 = jnp.pad(embedding_bias, (0, _VPAD - _V)).reshape(1, _VPAD)
    t2 = temperatures.reshape(b, 1)
    tp2 = top_ps.reshape(b, 1)
    tk2 = top_ks.reshape(b, 1)

    ids2, lp2 = pl.pallas_call(
        _body,
        grid=(_NBLK,),
        in_specs=[
            pl.BlockSpec((b, _D), lambda i: (0, 0)),
            pl.BlockSpec((1, _BLK), lambda i: (0, i)),
            pl.BlockSpec((b, 1), lambda i: (0, 0)),
            pl.BlockSpec((b, 1), lambda i: (0, 0)),
            pl.BlockSpec((b, 1), lambda i: (0, 0)),
            pl.BlockSpec((b, _VPAD), lambda i: (0, 0)),
            pl.BlockSpec((_BLK, _D), lambda i: (i, 0)),
        ],
        out_specs=[
            pl.BlockSpec((b, 1), lambda i: (0, 0)),
            pl.BlockSpec((b, 1), lambda i: (0, 0)),
        ],
        out_shape=[
            jax.ShapeDtypeStruct((b, 1), jnp.int32),
            jax.ShapeDtypeStruct((b, 1), jnp.float32),
        ],
        scratch_shapes=[
            pltpu.VMEM((b, _VPAD), jnp.float32),
            pltpu.VMEM((b, _VPAD), jnp.float32),
        ],
        compiler_params=pltpu.CompilerParams(
            dimension_semantics=("arbitrary",)),
    )(hidden_states, bias2, t2, tp2, tk2, g, embedding)
    return ids2.reshape(b), lp2.reshape(b)
